# Initial kernel scaffold; baseline (speedup 1.0000x reference)
import jax
import jax.numpy as jnp
from jax.experimental import pallas as pl


def kernel(means3d, covariances, opacities, features):
    raise NotImplementedError("write your pallas kernel here")


# speedup vs baseline: 2.0003x; 2.0003x over previous
import jax
import jax.numpy as jnp
from jax import lax
from jax.experimental import pallas as pl
from jax.experimental.pallas import tpu as pltpu

_VMIN = (-40.0, -40.0, -1.0)
_VMAX = (40.0, 40.0, 5.4)
_VOXEL = 0.4
_GRID = (200, 200, 16)
_PX = 202
_ROWS = _PX * _PX + 12
_C = 512


def _compute_weights(meta, wbuf, ibuf):
    m0 = meta[0, 0:1, :]
    m1 = meta[0, 1:2, :]
    m2 = meta[0, 2:3, :]
    v0 = meta[0, 3:4, :]
    v1 = meta[0, 4:5, :]
    v2 = meta[0, 5:6, :]
    o = meta[0, 6:7, :]

    keep = (o > 1e-6)
    keep &= (m0 >= _VMIN[0]) & (m0 <= _VMAX[0])
    keep &= (m1 >= _VMIN[1]) & (m1 <= _VMAX[1])
    keep &= (m2 >= _VMIN[2]) & (m2 <= _VMAX[2])

    bx = jnp.floor((m0 - _VMIN[0]) / _VOXEL).astype(jnp.int32)
    by = jnp.floor((m1 - _VMIN[1]) / _VOXEL).astype(jnp.int32)
    bz = jnp.floor((m2 - _VMIN[2]) / _VOXEL).astype(jnp.int32)

    iv0 = 1.0 / v0
    iv1 = 1.0 / v1
    iv2 = 1.0 / v2
    r0 = 3.0 * jnp.sqrt(v0)
    r1 = 3.0 * jnp.sqrt(v1)
    r2 = 3.0 * jnp.sqrt(v2)

    wks = []
    for k in range(27):
        dxo = k // 9 - 1
        dyo = (k // 3) % 3 - 1
        dzo = k % 3 - 1
        vxk = bx + dxo
        vyk = by + dyo
        vzk = bz + dzo
        inb = (vxk >= 0) & (vxk < _GRID[0])
        inb &= (vyk >= 0) & (vyk < _GRID[1])
        inb &= (vzk >= 0) & (vzk < _GRID[2])
        d0 = (_VMIN[0] + (vxk.astype(jnp.float32) + 0.5) * _VOXEL) - m0
        d1 = (_VMIN[1] + (vyk.astype(jnp.float32) + 0.5) * _VOXEL) - m1
        d2 = (_VMIN[2] + (vzk.astype(jnp.float32) + 0.5) * _VOXEL) - m2
        aabb = (jnp.abs(d0) <= r0) & (jnp.abs(d1) <= r1) & (jnp.abs(d2) <= r2)
        quad = d0 * d0 * iv0 + d1 * d1 * iv1 + d2 * d2 * iv2
        wk = o * jnp.exp(-0.5 * quad)
        wk = jnp.where(inb & aabb & keep, wk, 0.0)
        wks.append(wk)
    wks.extend([jnp.zeros((1, _C), jnp.float32)] * 5)
    wbuf[...] = jnp.concatenate(wks, axis=0).T

    bxc = jnp.clip(bx, 0, _GRID[0] - 1)
    byc = jnp.clip(by, 0, _GRID[1] - 1)
    bzc = jnp.clip(bz, 0, _GRID[2] - 1)
    imeta = jnp.concatenate(
        [bxc * _PX + byc, bzc] + [jnp.zeros((1, _C), jnp.int32)] * 6, axis=0
    )
    ibuf[...] = imeta.T


def _dens_kernel(meta, dens, wbuf, ibuf):
    i = pl.program_id(0)

    @pl.when(i == 0)
    def _init():
        dens[...] = jnp.zeros(dens.shape, dens.dtype)

    _compute_weights(meta, wbuf, ibuf)

    zi16 = lax.broadcasted_iota(jnp.int32, (3, 16), 1)
    ri16 = lax.broadcasted_iota(jnp.int32, (16, 16), 0)

    def body(n, carry):
        rb = ibuf[n, 0]
        z0 = ibuf[n, 1]
        for dxi in range(3):
            wz16 = jnp.zeros((3, 16), jnp.float32)
            for dzi in range(3):
                wcol = jnp.stack(
                    [
                        wbuf[n, dxi * 9 + 0 + dzi],
                        wbuf[n, dxi * 9 + 3 + dzi],
                        wbuf[n, dxi * 9 + 6 + dzi],
                    ]
                ).reshape(3, 1)
                wz16 = wz16 + jnp.where(zi16 == (z0 - 1 + dzi), wcol, 0.0)
            rs = rb + dxi * _PX
            rs8 = (rs // 8) * 8
            off = rs - rs8
            upd = jnp.zeros((16, 16), jnp.float32)
            for j in range(3):
                upd = upd + jnp.where(ri16 == off + j, wz16[j : j + 1, :], 0.0)
            dt = dens[pl.ds(rs8, 16), :]
            dens[pl.ds(rs8, 16), :] = dt + upd
        return carry

    lax.fori_loop(0, _C, body, 0)


def _feats_kernel(meta, ft, feats, wbuf, ibuf):
    i = pl.program_id(0)

    @pl.when(i == 0)
    def _init():
        feats[...] = jnp.zeros(feats.shape, feats.dtype)

    _compute_weights(meta, wbuf, ibuf)

    zl256 = lax.broadcasted_iota(jnp.int32, (3, 256), 1) // 16
    ri8 = lax.broadcasted_iota(jnp.int32, (8, 256), 0)
    ri16 = lax.broadcasted_iota(jnp.int32, (16, 256), 0)

    def body(n, carry):
        rb = ibuf[n, 0]
        z0 = ibuf[n, 1]
        n8 = (n // 8) * 8
        fblk = ft[pl.ds(n8, 8), :]
        fr = jnp.sum(jnp.where(ri8 == (n - n8), fblk, 0.0), axis=0, keepdims=True)
        for dxi in range(3):
            wz = jnp.zeros((3, 256), jnp.float32)
            for dzi in range(3):
                wcol = jnp.stack(
                    [
                        wbuf[n, dxi * 9 + 0 + dzi],
                        wbuf[n, dxi * 9 + 3 + dzi],
                        wbuf[n, dxi * 9 + 6 + dzi],
                    ]
                ).reshape(3, 1)
                wz = wz + jnp.where(zl256 == (z0 - 1 + dzi), wcol, 0.0)
            rs = rb + dxi * _PX
            rs8 = (rs // 8) * 8
            off = rs - rs8
            upd = jnp.zeros((16, 256), jnp.float32)
            for j in range(3):
                upd = upd + jnp.where(ri16 == off + j, wz[j : j + 1, :], 0.0)
            ftile = feats[pl.ds(rs8, 16), :]
            feats[pl.ds(rs8, 16), :] = ftile + upd * fr
        return carry

    lax.fori_loop(0, _C, body, 0)


def _norm_kernel(dens, feats, out):
    zl = lax.broadcasted_iota(jnp.int32, (16, 256), 1) // 16
    dt = dens[...]
    div = jnp.zeros((16, 256), jnp.float32)
    for z in range(16):
        dz = jnp.maximum(dt[:, z : z + 1], 1e-6)
        div = div + jnp.where(zl == z, dz, 0.0)
    out[...] = feats[...] / div


def kernel(means3d, covariances, opacities, features):
    N = means3d.shape[0]
    D = features.shape[1]
    G = (N + _C - 1) // _C
    npad = G * _C - N

    var = jnp.diagonal(covariances, axis1=-2, axis2=-1)
    meta = jnp.stack(
        [
            jnp.pad(means3d[:, 0], (0, npad)),
            jnp.pad(means3d[:, 1], (0, npad)),
            jnp.pad(means3d[:, 2], (0, npad)),
            jnp.pad(var[:, 0], (0, npad), constant_values=1.0),
            jnp.pad(var[:, 1], (0, npad), constant_values=1.0),
            jnp.pad(var[:, 2], (0, npad), constant_values=1.0),
            jnp.pad(opacities, (0, npad)),
            jnp.zeros((G * _C,), jnp.float32),
        ],
        axis=0,
    ).reshape(8, G, _C).transpose(1, 0, 2)
    ft = jnp.tile(jnp.pad(features, ((0, npad), (0, 0))), (1, 16))

    scatter_scratch = [
        pltpu.VMEM((_C, 32), jnp.float32),
        pltpu.VMEM((_C, 8), jnp.int32),
    ]

    dens_p = pl.pallas_call(
        _dens_kernel,
        grid=(G,),
        in_specs=[pl.BlockSpec((1, 8, _C), lambda i: (i, 0, 0))],
        out_specs=pl.BlockSpec((_ROWS, 16), lambda i: (0, 0)),
        out_shape=jax.ShapeDtypeStruct((_ROWS, 16), jnp.float32),
        scratch_shapes=scatter_scratch,
    )(meta)

    feats_p = pl.pallas_call(
        _feats_kernel,
        grid=(G,),
        in_specs=[
            pl.BlockSpec((1, 8, _C), lambda i: (i, 0, 0)),
            pl.BlockSpec((_C, 16 * D), lambda i: (i, 0)),
        ],
        out_specs=pl.BlockSpec((_ROWS, 16 * D), lambda i: (0, 0)),
        out_shape=jax.ShapeDtypeStruct((_ROWS, 16 * D), jnp.float32),
        scratch_shapes=scatter_scratch,
    )(meta, ft)

    feats_n = pl.pallas_call(
        _norm_kernel,
        grid=(_ROWS // 16,),
        in_specs=[
            pl.BlockSpec((16, 16), lambda i: (i, 0)),
            pl.BlockSpec((16, 16 * D), lambda i: (i, 0)),
        ],
        out_specs=pl.BlockSpec((16, 16 * D), lambda i: (i, 0)),
        out_shape=jax.ShapeDtypeStruct((_ROWS, 16 * D), jnp.float32),
    )(dens_p, feats_p)

    dens_p = dens_p[: _PX * _PX].reshape(_PX, _PX, 16)
    feats_n = feats_n[: _PX * _PX].reshape(_PX, _PX, 16, D)
    grid_density = dens_p[1:201, 1:201, :]
    grid_feats_norm = feats_n[1:201, 1:201, :, :]
    return (grid_density, grid_feats_norm)
